# baseline (speedup 1.0000x reference)
import functools

import jax
import jax.numpy as jnp
from jax import lax
from jax.experimental import pallas as pl
from jax.experimental.pallas import tpu as pltpu
from jax.experimental.pallas import tpu_sc as plsc

N = 16777216
SPARSITY = 0.01
SIGMOID_BIAS = 5.0
K = int(SPARSITY * N)

NC, NS, LANES = 2, 16, 16
NTILES = NC * NS
PER_TILE = N // NTILES
NBUF = 2
UNROLL = 8

B1, B2, B3 = 2048, 2048, 512

_mesh = plsc.VectorSubcoreMesh(core_axis_name="c", subcore_axis_name="s")
_sc_params = pltpu.CompilerParams(needs_layout_passes=False)


def _wid():
    return lax.axis_index("s") * NC + lax.axis_index("c")


def _zero_hist(hist_v, nwords):
    zeros = jnp.zeros((LANES,), jnp.int32)

    @plsc.parallel_loop(0, nwords // LANES, 1, unroll=UNROLL)
    def _(i):
        hist_v[pl.ds(i * LANES, LANES)] = zeros


_P1_BLK = 4096
_P1_NBLK = PER_TILE // _P1_BLK
_P1_NBUF = 4


@functools.partial(
    pl.kernel,
    out_type=[
        jax.ShapeDtypeStruct((N,), jnp.float32),
        jax.ShapeDtypeStruct((NTILES, LANES * B1), jnp.int32),
    ],
    mesh=_mesh,
    scratch_types=[
        pltpu.VMEM((_P1_NBUF, _P1_BLK), jnp.float32),
        pltpu.VMEM((_P1_NBUF, _P1_BLK), jnp.float32),
        pltpu.VMEM((_P1_NBUF, _P1_BLK), jnp.float32),
        pltpu.VMEM((LANES * B1,), jnp.int32),
        pltpu.SemaphoreType.DMA,
        pltpu.SemaphoreType.DMA,
        pltpu.SemaphoreType.DMA,
        pltpu.SemaphoreType.DMA,
        pltpu.SemaphoreType.DMA,
        pltpu.SemaphoreType.DMA,
        pltpu.SemaphoreType.DMA,
        pltpu.SemaphoreType.DMA,
        pltpu.SemaphoreType.DMA,
        pltpu.SemaphoreType.DMA,
        pltpu.SemaphoreType.DMA,
        pltpu.SemaphoreType.DMA,
    ],
    compiler_params=_sc_params,
)
def _p1(p_hbm, f_hbm, tv_hbm, hist_hbm, p_v, f_v, t_v, hist_v,
        sp0, sp1, sp2, sp3, sf0, sf1, sf2, sf3, ss0, ss1, ss2, ss3):
    wid = _wid()
    base = wid * PER_TILE
    sp = (sp0, sp1, sp2, sp3)
    sf = (sf0, sf1, sf2, sf3)
    ss = (ss0, ss1, ss2, ss3)
    for j in range(_P1_NBUF):
        off = base + j * _P1_BLK
        pltpu.async_copy(p_hbm.at[pl.ds(off, _P1_BLK)], p_v.at[j], sp[j])
        pltpu.async_copy(f_hbm.at[pl.ds(off, _P1_BLK)], f_v.at[j], sf[j])
    _zero_hist(hist_v, LANES * B1)
    lane = lax.iota(jnp.int32, LANES)
    ones = jnp.ones((LANES,), jnp.int32)

    def outer(g, _):
        for j in range(_P1_NBUF):
            blk = g * _P1_NBUF + j
            off = base + blk * _P1_BLK
            pltpu.make_async_copy(
                p_hbm.at[pl.ds(off, _P1_BLK)], p_v.at[j], sp[j]).wait()
            pltpu.make_async_copy(
                f_hbm.at[pl.ds(off, _P1_BLK)], f_v.at[j], sf[j]).wait()

            @pl.when(blk >= _P1_NBUF)
            def _wait_store():
                pltpu.make_async_copy(
                    t_v.at[j], tv_hbm.at[pl.ds(base, _P1_BLK)], ss[j]).wait()

            @plsc.parallel_loop(0, _P1_BLK // LANES, 1, unroll=UNROLL)
            def _vec(i):
                s = i * LANES
                t = f_v[j, pl.ds(s, LANES)] - p_v[j, pl.ds(s, LANES)]
                t_v[j, pl.ds(s, LANES)] = t
                bits = plsc.bitcast(jnp.abs(t), jnp.int32)
                bkt = lax.shift_right_logical(bits, 20)
                idx = lax.shift_left(bkt, 4) + lane
                plsc.addupdate_scatter(hist_v, [idx], ones)
            pltpu.async_copy(t_v.at[j], tv_hbm.at[pl.ds(off, _P1_BLK)], ss[j])

            @pl.when(blk + _P1_NBUF < _P1_NBLK)
            def _next_load():
                noff = base + (blk + _P1_NBUF) * _P1_BLK
                pltpu.async_copy(
                    p_hbm.at[pl.ds(noff, _P1_BLK)], p_v.at[j], sp[j])
                pltpu.async_copy(
                    f_hbm.at[pl.ds(noff, _P1_BLK)], f_v.at[j], sf[j])
        return 0

    lax.fori_loop(0, _P1_NBLK // _P1_NBUF, outer, 0)
    for j in range(_P1_NBUF):
        pltpu.make_async_copy(
            t_v.at[j], tv_hbm.at[pl.ds(base, _P1_BLK)], ss[j]).wait()
    pltpu.sync_copy(hist_v, hist_hbm.at[wid])


_H_NBUF = 4


def _make_masked_hist(shift_match, shift_bucket, bmask, nbuckets, blk):
    nblk = PER_TILE // blk

    @functools.partial(
        pl.kernel,
        out_type=jax.ShapeDtypeStruct((NTILES, LANES * nbuckets), jnp.int32),
        mesh=_mesh,
        scratch_types=[
            pltpu.VMEM((_H_NBUF, blk), jnp.float32),
            pltpu.VMEM((LANES,), jnp.int32),
            pltpu.VMEM((LANES * nbuckets,), jnp.int32),
            pltpu.SemaphoreType.DMA,
            pltpu.SemaphoreType.DMA,
            pltpu.SemaphoreType.DMA,
            pltpu.SemaphoreType.DMA,
        ],
        compiler_params=_sc_params,
    )
    def hist_pass(tv_hbm, sel_hbm, hist_hbm, tv_v, sel_v, hist_v,
                  s0, s1, s2, s3):
        wid = _wid()
        base = wid * PER_TILE
        sems = (s0, s1, s2, s3)
        for j in range(_H_NBUF):
            off = base + j * blk
            pltpu.async_copy(tv_hbm.at[pl.ds(off, blk)], tv_v.at[j], sems[j])
        pltpu.sync_copy(sel_hbm, sel_v)
        _zero_hist(hist_v, LANES * nbuckets)
        sel = sel_v[...]
        lane = lax.iota(jnp.int32, LANES)
        ones = jnp.ones((LANES,), jnp.int32)

        def outer(g, _):
            for j in range(_H_NBUF):
                b = g * _H_NBUF + j
                off = base + b * blk
                pltpu.make_async_copy(
                    tv_hbm.at[pl.ds(off, blk)], tv_v.at[j], sems[j]).wait()

                @plsc.parallel_loop(0, blk // LANES, 1, unroll=UNROLL)
                def _vec(i):
                    s = i * LANES
                    t = tv_v[j, pl.ds(s, LANES)]
                    bits = plsc.bitcast(jnp.abs(t), jnp.int32)
                    match = lax.shift_right_logical(bits, shift_match) == sel
                    bucket = jnp.bitwise_and(
                        lax.shift_right_logical(bits, shift_bucket), bmask)
                    idx = lax.shift_left(bucket, 4) + lane
                    plsc.addupdate_scatter(hist_v, [idx], ones, mask=match)

                @pl.when(b + _H_NBUF < nblk)
                def _next_load():
                    noff = base + (b + _H_NBUF) * blk
                    pltpu.async_copy(
                        tv_hbm.at[pl.ds(noff, blk)], tv_v.at[j], sems[j])
            return 0

        lax.fori_loop(0, nblk // _H_NBUF, outer, 0)
        pltpu.sync_copy(hist_v, hist_hbm.at[wid])

    return hist_pass


_p2 = _make_masked_hist(20, 9, B2 - 1, B2, 16384)
_p3 = _make_masked_hist(9, 0, B3 - 1, B3, 16384)


def _select(hist, rank, prev, nbuckets, nbits, last=False):
    def body(h_ref, r_ref, p_ref, selv_ref, sels_ref, rn_ref, *prop_ref):
        h = jnp.sum(h_ref[...], axis=0, keepdims=True)
        col = lax.shift_right_logical(
            lax.broadcasted_iota(jnp.int32, h.shape, 1), 4)
        k = r_ref[0, 0]

        def step(_, lohi):
            lo, hi = lohi
            mid = (lo + hi) // 2
            s = jnp.sum(jnp.where(col >= mid, h, 0))
            ok = s >= k
            return jnp.where(ok, mid, lo), jnp.where(ok, hi, mid)

        lo, _hi = lax.fori_loop(
            0, nbits, step, (jnp.int32(0), jnp.int32(nbuckets)))
        above = jnp.sum(jnp.where(col >= lo + 1, h, 0))
        combined = (p_ref[0, 0] << nbits) | lo
        rnext = k - above
        selv_ref[...] = jnp.full((LANES,), combined, jnp.int32)
        sels_ref[0, 0] = combined
        rn_ref[0, 0] = rnext
        if last:
            cnt_above = K - rnext
            prop_ref[0][0, 0] = (
                (5 * (2 * cnt_above - N)).astype(jnp.float32) / jnp.float32(N))

    out_specs = [
        pl.BlockSpec(memory_space=pltpu.VMEM),
        pl.BlockSpec(memory_space=pltpu.SMEM),
        pl.BlockSpec(memory_space=pltpu.SMEM),
    ]
    out_shape = [
        jax.ShapeDtypeStruct((LANES,), jnp.int32),
        jax.ShapeDtypeStruct((1, 1), jnp.int32),
        jax.ShapeDtypeStruct((1, 1), jnp.int32),
    ]
    if last:
        out_specs.append(pl.BlockSpec(memory_space=pltpu.SMEM))
        out_shape.append(jax.ShapeDtypeStruct((1, 1), jnp.float32))
    return pl.pallas_call(
        body,
        in_specs=[
            pl.BlockSpec(memory_space=pltpu.VMEM),
            pl.BlockSpec(memory_space=pltpu.SMEM),
            pl.BlockSpec(memory_space=pltpu.SMEM),
        ],
        out_specs=out_specs,
        out_shape=out_shape,
    )(hist, rank, prev)


_SIG_HI = 0.9933071490757153
_SIG_LO = 0.0066928509242848554

_P4_BLK = 8192
_P4_NBLK = PER_TILE // _P4_BLK
_P4_NBUF = 4


@functools.partial(
    pl.kernel,
    out_type=[
        jax.ShapeDtypeStruct((N,), jnp.float32),
        jax.ShapeDtypeStruct((N,), jnp.float32),
    ],
    mesh=_mesh,
    scratch_types=[
        pltpu.VMEM((_P4_NBUF, _P4_BLK), jnp.float32),
        pltpu.VMEM((_P4_NBUF, _P4_BLK), jnp.float32),
        pltpu.VMEM((_P4_NBUF, _P4_BLK), jnp.float32),
        pltpu.VMEM((LANES,), jnp.int32),
        pltpu.SemaphoreType.DMA,
        pltpu.SemaphoreType.DMA,
        pltpu.SemaphoreType.DMA,
        pltpu.SemaphoreType.DMA,
        pltpu.SemaphoreType.DMA,
        pltpu.SemaphoreType.DMA,
        pltpu.SemaphoreType.DMA,
        pltpu.SemaphoreType.DMA,
        pltpu.SemaphoreType.DMA,
        pltpu.SemaphoreType.DMA,
        pltpu.SemaphoreType.DMA,
        pltpu.SemaphoreType.DMA,
    ],
    compiler_params=_sc_params,
)
def _p4(tv_hbm, thr_hbm, mask_hbm, delta_hbm, tv_v, m_v, d_v, thr_v,
        sl0, sl1, sl2, sl3, sm0, sm1, sm2, sm3, sd0, sd1, sd2, sd3):
    wid = _wid()
    base = wid * PER_TILE
    sl = (sl0, sl1, sl2, sl3)
    sm = (sm0, sm1, sm2, sm3)
    sd = (sd0, sd1, sd2, sd3)
    for j in range(_P4_NBUF):
        off = base + j * _P4_BLK
        pltpu.async_copy(tv_hbm.at[pl.ds(off, _P4_BLK)], tv_v.at[j], sl[j])
    pltpu.sync_copy(thr_hbm, thr_v)
    thr = plsc.bitcast(thr_v[...], jnp.float32)
    hi = jnp.full((LANES,), SIGMOID_BIAS, jnp.float32)
    lo = jnp.full((LANES,), -SIGMOID_BIAS, jnp.float32)
    shi = jnp.full((LANES,), _SIG_HI, jnp.float32)
    slo = jnp.full((LANES,), _SIG_LO, jnp.float32)

    def outer(g, _):
        for j in range(_P4_NBUF):
            blk = g * _P4_NBUF + j
            off = base + blk * _P4_BLK
            pltpu.make_async_copy(
                tv_hbm.at[pl.ds(off, _P4_BLK)], tv_v.at[j], sl[j]).wait()

            @pl.when(blk >= _P4_NBUF)
            def _wait_stores():
                pltpu.make_async_copy(
                    m_v.at[j], mask_hbm.at[pl.ds(base, _P4_BLK)], sm[j]).wait()
                pltpu.make_async_copy(
                    d_v.at[j], delta_hbm.at[pl.ds(base, _P4_BLK)], sd[j]).wait()

            @plsc.parallel_loop(0, _P4_BLK // LANES, 1, unroll=UNROLL)
            def _vec(i):
                s = i * LANES
                t = tv_v[j, pl.ds(s, LANES)]
                above = jnp.abs(t) > thr
                m_v[j, pl.ds(s, LANES)] = jnp.where(above, hi, lo)
                d_v[j, pl.ds(s, LANES)] = t * jnp.where(above, shi, slo)

            pltpu.async_copy(m_v.at[j], mask_hbm.at[pl.ds(off, _P4_BLK)], sm[j])
            pltpu.async_copy(d_v.at[j], delta_hbm.at[pl.ds(off, _P4_BLK)], sd[j])

            @pl.when(blk + _P4_NBUF < _P4_NBLK)
            def _next_load():
                noff = base + (blk + _P4_NBUF) * _P4_BLK
                pltpu.async_copy(
                    tv_hbm.at[pl.ds(noff, _P4_BLK)], tv_v.at[j], sl[j])
        return 0

    lax.fori_loop(0, _P4_NBLK // _P4_NBUF, outer, 0)
    for j in range(_P4_NBUF):
        pltpu.make_async_copy(
            m_v.at[j], mask_hbm.at[pl.ds(base, _P4_BLK)], sm[j]).wait()
        pltpu.make_async_copy(
            d_v.at[j], delta_hbm.at[pl.ds(base, _P4_BLK)], sd[j]).wait()


def kernel(pretrained, finetuned):
    tv, h1 = _p1(pretrained, finetuned)

    k0 = jnp.full((1, 1), K, jnp.int32)
    zero = jnp.zeros((1, 1), jnp.int32)
    sel1v, j1s, r2 = _select(h1, k0, zero, B1, 11)
    h2 = _p2(tv, sel1v)
    sel12v, j12s, r3 = _select(h2, r2, j1s, B2, 11)
    h3 = _p3(tv, sel12v)
    thrv, _bits, _r4, prop11 = _select(h3, r3, j12s, B3, 9, last=True)

    mask, delta = _p4(tv, thrv)
    return (delta, mask, prop11[0, 0])
